# baseline (speedup 1.0000x reference)
import functools

import jax
import jax.numpy as jnp
from jax import lax
from jax.experimental import pallas as pl
from jax.experimental.pallas import tpu as pltpu
from jax.experimental.pallas import tpu_sc as plsc

_C = 10
_N = 1048576
_LANES = 16
_NW = 32
_RW = _N // _NW
_R = 256
_NCHUNK = _RW // _R
_G = _R // _LANES
_BINS = 128

_mesh = plsc.VectorSubcoreMesh(core_axis_name="c", subcore_axis_name="s")


@functools.partial(
    pl.kernel,
    out_type=jax.ShapeDtypeStruct((_NW, _BINS), jnp.int32),
    mesh=_mesh,
    compiler_params=pltpu.CompilerParams(
        needs_layout_passes=False, use_tc_tiling_on_sc=True),
    scratch_types=[
        pltpu.VMEM((_R, _C), jnp.float32),
        pltpu.VMEM((_R,), jnp.int32),
        pltpu.VMEM((_LANES * _BINS,), jnp.int32),
        pltpu.SemaphoreType.DMA,
        pltpu.SemaphoreType.DMA,
    ],
)
def _confusion(yp_hbm, yt_hbm, out_hbm, ybuf, tbuf, cmbuf, sp, st):
    wid = lax.axis_index("s") * 2 + lax.axis_index("c")
    base = wid * _RW

    iota = lax.iota(jnp.int32, _LANES)
    zero = jnp.zeros((_LANES,), jnp.int32)
    ones = jnp.ones((_LANES,), jnp.int32)
    lane_off = iota * _BINS
    cols = [jnp.full((_LANES,), c, jnp.int32) for c in range(_C)]

    for j in range(_BINS):
        cmbuf[pl.ds(j * _LANES, _LANES)] = zero

    def chunk_body(i, carry):
        off = base + i * _R
        pltpu.make_async_copy(yp_hbm.at[pl.ds(off, _R)], ybuf, sp).start()
        pltpu.make_async_copy(yt_hbm.at[pl.ds(off, _R)], tbuf, st).start()
        pltpu.make_async_copy(yp_hbm.at[pl.ds(off, _R)], ybuf, sp).wait()
        pltpu.make_async_copy(yt_hbm.at[pl.ds(off, _R)], tbuf, st).wait()

        def body(g, c2):
            rowb = g * _LANES + iota
            t = plsc.load_gather(tbuf, [rowb])
            vs = [plsc.load_gather(ybuf, [rowb, cols[c]]) for c in range(_C)]
            cands = [(v, jnp.full((_LANES,), c, jnp.int32))
                     for c, v in enumerate(vs)]
            while len(cands) > 1:
                nxt = []
                for k in range(0, len(cands) - 1, 2):
                    (va, pa), (vb, pb) = cands[k], cands[k + 1]
                    gt = vb > va
                    nxt.append((jnp.where(gt, vb, va), jnp.where(gt, pb, pa)))
                if len(cands) % 2:
                    nxt.append(cands[-1])
                cands = nxt
            p = cands[0][1]
            plsc.addupdate_scatter(cmbuf, [lane_off + (t * _C + p)], ones)
            return c2

        lax.fori_loop(0, _G, body, 0, unroll=4)
        return carry

    lax.fori_loop(0, _NCHUNK, chunk_body, 0)

    half = _LANES // 2
    while half >= 1:
        for l in range(half):
            for j in range(_BINS // _LANES):
                a = l * _BINS + j * _LANES
                bb = (l + half) * _BINS + j * _LANES
                cmbuf[pl.ds(a, _LANES)] = (
                    cmbuf[pl.ds(a, _LANES)] + cmbuf[pl.ds(bb, _LANES)])
        half //= 2

    pltpu.sync_copy(cmbuf.at[pl.ds(0, _BINS)], out_hbm.at[wid])


def kernel(y_pred, y_true):
    yt = y_true.reshape(-1).astype(jnp.int32)
    parts = _confusion(y_pred, yt)
    counts = parts.sum(axis=0)[: _C * _C].reshape(_C, _C)
    cm = counts.astype(jnp.float32)
    hist_true = cm.sum(axis=1)
    hist_pred = cm.sum(axis=0)
    cmn = cm / cm.sum()
    expected = jnp.outer(hist_true, hist_pred)
    expected = expected / expected.sum()
    i = jnp.arange(_C, dtype=jnp.float32)
    weight_matrix = (i[:, None] - i[None, :]) ** 2
    return 1.0 - (weight_matrix * cmn).sum() / (weight_matrix * expected).sum()


# speedup vs baseline: 1.3771x; 1.3771x over previous
import functools

import jax
import jax.numpy as jnp
from jax import lax
from jax.experimental import pallas as pl
from jax.experimental.pallas import tpu as pltpu
from jax.experimental.pallas import tpu_sc as plsc

_C = 10
_N = 1048576
_LANES = 16
_NW = 32
_RW = _N // _NW
_R = 256
_NCHUNK = _RW // _R
_G = _R // _LANES
_BINS = 128

_mesh = plsc.VectorSubcoreMesh(core_axis_name="c", subcore_axis_name="s")


@functools.partial(
    pl.kernel,
    out_type=jax.ShapeDtypeStruct((_NW, _BINS), jnp.int32),
    mesh=_mesh,
    compiler_params=pltpu.CompilerParams(
        needs_layout_passes=False, use_tc_tiling_on_sc=True),
    scratch_types=[
        pltpu.VMEM((_R, _C), jnp.float32),
        pltpu.VMEM((_R, _C), jnp.float32),
        pltpu.VMEM((_R,), jnp.int32),
        pltpu.VMEM((_R,), jnp.int32),
        pltpu.VMEM((_LANES * _BINS,), jnp.int32),
        pltpu.SemaphoreType.DMA,
        pltpu.SemaphoreType.DMA,
        pltpu.SemaphoreType.DMA,
        pltpu.SemaphoreType.DMA,
    ],
)
def _confusion(yp_hbm, yt_hbm, out_hbm, ybuf0, ybuf1, tbuf0, tbuf1, cmbuf,
               sp0, sp1, st0, st1):
    wid = lax.axis_index("s") * 2 + lax.axis_index("c")
    base = wid * _RW

    iota = lax.iota(jnp.int32, _LANES)
    zero = jnp.zeros((_LANES,), jnp.int32)
    ones = jnp.ones((_LANES,), jnp.int32)
    lane_off = iota * _BINS
    cols = [jnp.full((_LANES,), c, jnp.int32) for c in range(_C)]

    for j in range(_BINS):
        cmbuf[pl.ds(j * _LANES, _LANES)] = zero

    def start(off, yb, tb, semp, semt):
        pltpu.make_async_copy(yp_hbm.at[pl.ds(off, _R)], yb, semp).start()
        pltpu.make_async_copy(yt_hbm.at[pl.ds(off, _R)], tb, semt).start()

    def wait(yb, tb, semp, semt):
        pltpu.make_async_copy(yp_hbm.at[pl.ds(base, _R)], yb, semp).wait()
        pltpu.make_async_copy(yt_hbm.at[pl.ds(base, _R)], tb, semt).wait()

    def compute(yb, tb):
        def body(g, c2):
            rowb = g * _LANES + iota
            t = plsc.load_gather(tb, [rowb])
            vs = [plsc.load_gather(yb, [rowb, cols[c]]) for c in range(_C)]
            cands = [(v, jnp.full((_LANES,), c, jnp.int32))
                     for c, v in enumerate(vs)]
            while len(cands) > 1:
                nxt = []
                for k in range(0, len(cands) - 1, 2):
                    (va, pa), (vb, pb) = cands[k], cands[k + 1]
                    gt = vb > va
                    nxt.append((jnp.where(gt, vb, va), jnp.where(gt, pb, pa)))
                if len(cands) % 2:
                    nxt.append(cands[-1])
                cands = nxt
            p = cands[0][1]
            plsc.addupdate_scatter(cmbuf, [lane_off + (t * _C + p)], ones)
            return c2

        lax.fori_loop(0, _G, body, 0, unroll=4)

    start(base, ybuf0, tbuf0, sp0, st0)
    start(base + _R, ybuf1, tbuf1, sp1, st1)

    def chunk_pair(k, carry):
        wait(ybuf0, tbuf0, sp0, st0)
        compute(ybuf0, tbuf0)

        @pl.when(2 * k + 2 < _NCHUNK)
        def _():
            start(base + (2 * k + 2) * _R, ybuf0, tbuf0, sp0, st0)

        wait(ybuf1, tbuf1, sp1, st1)
        compute(ybuf1, tbuf1)

        @pl.when(2 * k + 3 < _NCHUNK)
        def _():
            start(base + (2 * k + 3) * _R, ybuf1, tbuf1, sp1, st1)

        return carry

    lax.fori_loop(0, _NCHUNK // 2, chunk_pair, 0)

    half = _LANES // 2
    while half >= 1:
        for l in range(half):
            for j in range(_BINS // _LANES):
                a = l * _BINS + j * _LANES
                bb = (l + half) * _BINS + j * _LANES
                cmbuf[pl.ds(a, _LANES)] = (
                    cmbuf[pl.ds(a, _LANES)] + cmbuf[pl.ds(bb, _LANES)])
        half //= 2

    pltpu.sync_copy(cmbuf.at[pl.ds(0, _BINS)], out_hbm.at[wid])


def kernel(y_pred, y_true):
    yt = y_true.reshape(-1).astype(jnp.int32)
    parts = _confusion(y_pred, yt)
    counts = parts.sum(axis=0)[: _C * _C].reshape(_C, _C)
    cm = counts.astype(jnp.float32)
    hist_true = cm.sum(axis=1)
    hist_pred = cm.sum(axis=0)
    cmn = cm / cm.sum()
    expected = jnp.outer(hist_true, hist_pred)
    expected = expected / expected.sum()
    i = jnp.arange(_C, dtype=jnp.float32)
    weight_matrix = (i[:, None] - i[None, :]) ** 2
    return 1.0 - (weight_matrix * cmn).sum() / (weight_matrix * expected).sum()
